# baseline (speedup 1.0000x reference)
import jax
import jax.numpy as jnp
from jax import lax
from jax.experimental import pallas as pl
from jax.experimental.pallas import tpu as pltpu
from jax.experimental.pallas import tpu_sc as plsc

B = 1024
L = 200
PP = 208
D = 64
GAMMA = 0.7
NW = 32
ROWS_PER_W = (2 * B) // NW
GL = 80
MAX_KCH = GL // 16


def _sc_body(rperm_hbm, sp_hbm, lens_hbm, emb_hbm, out_hbm,
             sp_v, perm_v, lens_v, idsb_v, bufb_v, out_v, embv_v, tmp_v,
             sem, *, mask_id):
    nc = 2
    wid = lax.axis_index("s") * nc + lax.axis_index("c")
    base = wid * ROWS_PER_W

    pltpu.sync_copy(sp_hbm.at[pl.ds(base, ROWS_PER_W), :], sp_v)
    pltpu.sync_copy(rperm_hbm.at[pl.ds(base, ROWS_PER_W), :], perm_v)
    pltpu.sync_copy(lens_hbm.at[pl.ds(base, ROWS_PER_W), :], lens_v)
    pltpu.sync_copy(emb_hbm.at[mask_id], embv_v)

    vfill = jnp.full((16,), mask_id, jnp.int32)
    zerov = jnp.zeros((16,), jnp.int32)
    ss_v = tmp_v
    ss_v[0, pl.ds(0, 16)] = zerov

    def _scan16(x):
        cs = x
        for k in (1, 2, 4, 8):
            ss_v[0, pl.ds(16, 16)] = cs
            cs = cs + ss_v[0, pl.ds(16 - k, 16)]
        return cs

    def row_body(j, _):
        nvec = lens_v[j, pl.ds(0, 16)]
        keptv = lens_v[j, pl.ds(16, 16)]

        def wbody(c, cums):
            pvec = perm_v[j, pl.ds(c * 16, 16)]
            ind = pvec < nvec
            cs = _scan16(jnp.where(ind, 1, 0))
            cum = cs + jnp.full((16,), cums, jnp.int32)
            keep = ind & (cum <= keptv)
            ids = jnp.where(keep, sp_v[j, pl.ds(c * 16, 16)], vfill)
            idsb_v[pl.ds(c * 16, 16)] = ids
            ss_v[1, pl.ds(0, 16)] = cs
            return cums + ss_v[1, pl.ds(0, 16)][15]

        lax.fori_loop(0, PP // 16, wbody, jnp.int32(0))

        h = PP // 2
        pltpu.async_copy(emb_hbm.at[idsb_v.at[pl.ds(0, h)]],
                         bufb_v.at[pl.ds(0, h)], sem)
        pltpu.async_copy(emb_hbm.at[idsb_v.at[pl.ds(h, h)]],
                         bufb_v.at[pl.ds(h, h)], sem)
        pltpu.make_async_copy(emb_hbm.at[pl.ds(0, h)],
                              bufb_v.at[pl.ds(0, h)], sem).wait()
        pltpu.make_async_copy(emb_hbm.at[pl.ds(0, h)],
                              bufb_v.at[pl.ds(0, h)], sem).wait()

        def accum(c, accs):
            out = list(accs)
            for r in range(16):
                for k in range(4):
                    out[k] = out[k] + bufb_v[c * 16 + r, pl.ds(k * 16, 16)]
            return tuple(out)

        zero = jnp.zeros((16,), jnp.float32)
        accs = lax.fori_loop(0, PP // 16, accum, (zero, zero, zero, zero))

        extrav = (jnp.full((16,), PP, jnp.int32)
                  - keptv).astype(jnp.float32)
        for k in range(4):
            out_v[j, pl.ds(k * 16, 16)] = (
                accs[k] - extrav * embv_v[pl.ds(k * 16, 16)])
        return 0

    lax.fori_loop(0, ROWS_PER_W, row_body, 0)
    pltpu.sync_copy(out_v, out_hbm.at[pl.ds(base, ROWS_PER_W), :])


def _run_sc(perm, seqperm, lens2, item_emb, mask_id):
    import functools
    mesh = plsc.VectorSubcoreMesh(core_axis_name="c", subcore_axis_name="s",
                                  num_cores=2, num_subcores=16)
    return pl.kernel(
        functools.partial(_sc_body, mask_id=mask_id),
        out_type=jax.ShapeDtypeStruct((2 * B, D), jnp.float32),
        mesh=mesh,
        compiler_params=pltpu.CompilerParams(use_tc_tiling_on_sc=False),
        scratch_types=[
            pltpu.VMEM((ROWS_PER_W, PP), jnp.int32),
            pltpu.VMEM((ROWS_PER_W, PP), jnp.int32),
            pltpu.VMEM((ROWS_PER_W, 48), jnp.int32),
            pltpu.VMEM((PP,), jnp.int32),
            pltpu.VMEM((PP, D), jnp.float32),
            pltpu.VMEM((ROWS_PER_W, D), jnp.float32),
            pltpu.VMEM((D,), jnp.float32),
            pltpu.VMEM((PP // 16 + 1, 32), jnp.int32),
            pltpu.SemaphoreType.DMA,
        ],
    )(perm, seqperm, lens2, item_emb)


def _loss_kernel(sums_ref, corr_ref, lenb_ref, out_ref):
    rep = (sums_ref[...] + corr_ref[...]) / lenb_ref[...]
    ri = rep[:B, :]
    rj = rep[B:, :]
    dn = (((1,), (1,)), ((), ()))
    sim_ij = lax.dot_general(ri, rj, dn, preferred_element_type=jnp.float32)
    sim_ii = lax.dot_general(ri, ri, dn, preferred_element_type=jnp.float32)
    row = lax.broadcasted_iota(jnp.int32, (B, B), 0)
    col = lax.broadcasted_iota(jnp.int32, (B, B), 1)
    diag = row == col
    sim_ii = jnp.where(diag, -1e9, sim_ii)
    pos = jnp.sum(jnp.where(diag, sim_ij, 0.0), axis=1)
    m = jnp.maximum(jnp.max(sim_ij, axis=1), jnp.max(sim_ii, axis=1))
    z = (jnp.sum(jnp.exp(sim_ij - m[:, None]), axis=1)
         + jnp.sum(jnp.exp(sim_ii - m[:, None]), axis=1))
    logz = m + jnp.log(z)
    out_ref[...] = jnp.reshape(jnp.mean(logz - pos), (1, 1))


def _run_loss(sums, corr, lenb):
    return pl.pallas_call(
        _loss_kernel,
        out_shape=jax.ShapeDtypeStruct((1, 1), jnp.float32),
    )(sums, corr, lenb)


def _perm_const():
    def draw(key):
        keys = jax.random.split(key, B)
        return jax.vmap(lambda k: jax.random.uniform(k, (L,)))(keys)

    s = jnp.concatenate([draw(jax.random.key(123)),
                         draw(jax.random.key(456))], axis=0)
    perm = jnp.argsort(s, axis=1, stable=True).astype(jnp.int32)
    return jnp.pad(perm[:, ::-1], ((0, 0), (0, PP - L)),
                   constant_values=255)


def kernel(sequences, seq_lens, item_emb):
    v = int(item_emb.shape[0] - 1)
    perm = _perm_const()

    seq = sequences.astype(jnp.int32)
    lens2 = jnp.concatenate([seq_lens, seq_lens]).astype(jnp.int32)
    lf = lens2.astype(jnp.float32)
    sub = jnp.floor(jnp.float32(GAMMA) * lf).astype(jnp.int32)
    kept2 = lens2 - sub
    cumv = jnp.cumsum((perm < lens2[:, None]).astype(jnp.int32), axis=1)
    walked2 = 1 + jnp.argmax(cumv >= kept2[:, None], axis=1).astype(
        jnp.int32) // 16
    lensaux = jnp.concatenate(
        [jnp.broadcast_to(lens2[:, None], (2 * B, 16)),
         jnp.broadcast_to(kept2[:, None], (2 * B, 16)),
         jnp.broadcast_to(walked2[:, None], (2 * B, 16))], axis=1)
    seq2 = jnp.concatenate([seq, seq], axis=0)
    seqperm = jnp.take_along_axis(seq2, jnp.minimum(perm, L - 1), axis=1)

    sums = _run_sc(perm, seqperm, lensaux, item_emb, v)

    corr = sub.astype(jnp.float32)[:, None] * item_emb[v][None, :]
    lenb = jnp.broadcast_to(lf[:, None], (2 * B, D))

    loss = _run_loss(sums, corr, lenb)
    return jnp.reshape(loss, ())


# speedup vs baseline: 4.2913x; 4.2913x over previous
import jax
import jax.numpy as jnp
from jax import lax
from jax.experimental import pallas as pl
from jax.experimental.pallas import tpu as pltpu
from jax.experimental.pallas import tpu_sc as plsc

B = 1024
L = 200
PP = 208
D = 64
GAMMA = 0.7
NW = 32
ROWS_PER_W = (2 * B) // NW
GL = 80
MAX_KCH = GL // 16


def _sc_body(rperm_hbm, sp_hbm, lens_hbm, emb_hbm, out_hbm,
             sp_v, perm_v, lens_v, buf_v, out_v, embv_v, tmp_v, sem, *,
             mask_id):
    nc = 2
    wid = lax.axis_index("s") * nc + lax.axis_index("c")
    base = wid * ROWS_PER_W

    pltpu.sync_copy(sp_hbm.at[pl.ds(base, ROWS_PER_W), :], sp_v)
    pltpu.sync_copy(rperm_hbm.at[pl.ds(base, ROWS_PER_W), :], perm_v)
    pltpu.sync_copy(lens_hbm.at[pl.ds(base, ROWS_PER_W), :], lens_v)
    pltpu.sync_copy(emb_hbm.at[mask_id], embv_v)

    vfill = jnp.full((16,), mask_id, jnp.int32)
    zerov = jnp.zeros((16,), jnp.int32)
    ss_v = tmp_v
    ss_v[0, pl.ds(0, 16)] = zerov

    def _scan16(x):
        cs = x
        for k in (1, 2, 4, 8):
            ss_v[0, pl.ds(16, 16)] = cs
            cs = cs + ss_v[0, pl.ds(16 - k, 16)]
        return cs

    def row_body(j, _):
        nvec = lens_v[j, pl.ds(0, 16)]
        keptv = lens_v[j, pl.ds(16, 16)]
        kept = keptv[0]
        walked = lens_v[j, pl.ds(32, 16)][0]

        def wbody(c, cums):
            pvec = perm_v[j, pl.ds(c * 16, 16)]
            ind = pvec < nvec
            cs = _scan16(jnp.where(ind, 1, 0))
            cum = cs + jnp.full((16,), cums, jnp.int32)
            keep = ind & (cum <= keptv)
            ids = jnp.where(keep, sp_v[j, pl.ds(c * 16, 16)], vfill)
            pltpu.async_copy(emb_hbm.at[ids], buf_v.at[c], sem)
            ss_v[1, pl.ds(0, 16)] = cs
            return cums + ss_v[1, pl.ds(0, 16)][15]

        lax.fori_loop(0, walked, wbody, jnp.int32(0))

        def drain(c, _):
            pltpu.make_async_copy(emb_hbm.at[pl.ds(0, 16)], buf_v.at[0],
                                  sem).wait()
            return 0

        lax.fori_loop(0, walked, drain, 0)

        def accum(c, accs):
            out = list(accs)
            for r in range(16):
                for k in range(4):
                    out[k] = out[k] + buf_v[c, r, pl.ds(k * 16, 16)]
            return tuple(out)

        zero = jnp.zeros((16,), jnp.float32)
        accs = lax.fori_loop(0, walked, accum, (zero, zero, zero, zero))

        extrav = (jnp.full((16,), walked * 16, jnp.int32)
                  - keptv).astype(jnp.float32)
        for k in range(4):
            out_v[j, pl.ds(k * 16, 16)] = (
                accs[k] - extrav * embv_v[pl.ds(k * 16, 16)])
        return 0

    lax.fori_loop(0, ROWS_PER_W, row_body, 0)
    pltpu.sync_copy(out_v, out_hbm.at[pl.ds(base, ROWS_PER_W), :])


def _run_sc(perm, seqperm, lens2, item_emb, mask_id):
    import functools
    mesh = plsc.VectorSubcoreMesh(core_axis_name="c", subcore_axis_name="s",
                                  num_cores=2, num_subcores=16)
    return pl.kernel(
        functools.partial(_sc_body, mask_id=mask_id),
        out_type=jax.ShapeDtypeStruct((2 * B, D), jnp.float32),
        mesh=mesh,
        compiler_params=pltpu.CompilerParams(use_tc_tiling_on_sc=False),
        scratch_types=[
            pltpu.VMEM((ROWS_PER_W, PP), jnp.int32),
            pltpu.VMEM((ROWS_PER_W, PP), jnp.int32),
            pltpu.VMEM((ROWS_PER_W, 48), jnp.int32),
            pltpu.VMEM((PP // 16, 16, D), jnp.float32),
            pltpu.VMEM((ROWS_PER_W, D), jnp.float32),
            pltpu.VMEM((D,), jnp.float32),
            pltpu.VMEM((PP // 16 + 1, 32), jnp.int32),
            pltpu.SemaphoreType.DMA,
        ],
    )(perm, seqperm, lens2, item_emb)


def _loss_kernel(sums_ref, corr_ref, lenb_ref, out_ref):
    rep = (sums_ref[...] + corr_ref[...]) / lenb_ref[...]
    ri = rep[:B, :]
    rj = rep[B:, :]
    dn = (((1,), (1,)), ((), ()))
    sim_ij = lax.dot_general(ri, rj, dn, preferred_element_type=jnp.float32)
    sim_ii = lax.dot_general(ri, ri, dn, preferred_element_type=jnp.float32)
    row = lax.broadcasted_iota(jnp.int32, (B, B), 0)
    col = lax.broadcasted_iota(jnp.int32, (B, B), 1)
    diag = row == col
    sim_ii = jnp.where(diag, -1e9, sim_ii)
    pos = jnp.sum(jnp.where(diag, sim_ij, 0.0), axis=1)
    m = jnp.maximum(jnp.max(sim_ij, axis=1), jnp.max(sim_ii, axis=1))
    z = (jnp.sum(jnp.exp(sim_ij - m[:, None]), axis=1)
         + jnp.sum(jnp.exp(sim_ii - m[:, None]), axis=1))
    logz = m + jnp.log(z)
    out_ref[...] = jnp.reshape(jnp.mean(logz - pos), (1, 1))


def _run_loss(sums, corr, lenb):
    return pl.pallas_call(
        _loss_kernel,
        out_shape=jax.ShapeDtypeStruct((1, 1), jnp.float32),
    )(sums, corr, lenb)


def _perm_const():
    def draw(key):
        keys = jax.random.split(key, B)
        return jax.vmap(lambda k: jax.random.uniform(k, (L,)))(keys)

    s = jnp.concatenate([draw(jax.random.key(123)),
                         draw(jax.random.key(456))], axis=0)
    perm = jnp.argsort(s, axis=1, stable=True).astype(jnp.int32)
    return jnp.pad(perm[:, ::-1], ((0, 0), (0, PP - L)),
                   constant_values=255)


def kernel(sequences, seq_lens, item_emb):
    v = int(item_emb.shape[0] - 1)
    perm = _perm_const()

    seq = sequences.astype(jnp.int32)
    lens2 = jnp.concatenate([seq_lens, seq_lens]).astype(jnp.int32)
    lf = lens2.astype(jnp.float32)
    sub = jnp.floor(jnp.float32(GAMMA) * lf).astype(jnp.int32)
    kept2 = lens2 - sub
    cumv = jnp.cumsum((perm < lens2[:, None]).astype(jnp.int32), axis=1)
    walked2 = 1 + jnp.argmax(cumv >= kept2[:, None], axis=1).astype(
        jnp.int32) // 16
    lensaux = jnp.concatenate(
        [jnp.broadcast_to(lens2[:, None], (2 * B, 16)),
         jnp.broadcast_to(kept2[:, None], (2 * B, 16)),
         jnp.broadcast_to(walked2[:, None], (2 * B, 16))], axis=1)
    seq2 = jnp.concatenate([seq, seq], axis=0)
    seqperm = jnp.take_along_axis(seq2, jnp.minimum(perm, L - 1), axis=1)

    sums = _run_sc(perm, seqperm, lensaux, item_emb, v)

    corr = sub.astype(jnp.float32)[:, None] * item_emb[v][None, :]
    lenb = jnp.broadcast_to(lf[:, None], (2 * B, D))

    loss = _run_loss(sums, corr, lenb)
    return jnp.reshape(loss, ())
